# Initial kernel scaffold; baseline (speedup 1.0000x reference)
import jax
import jax.numpy as jnp
from jax.experimental import pallas as pl


def kernel(self_etab, etab, E_idx, seqs, x_mask, ln_gamma, ln_beta):
    raise NotImplementedError("write your pallas kernel here")


# speedup vs baseline: 1.8622x; 1.8622x over previous
import functools

import jax
import jax.numpy as jnp
from jax import lax
from jax.experimental import pallas as pl
from jax.experimental.pallas import tpu as pltpu
from jax.experimental.pallas import tpu_sc as plsc

_KO = 32


def _sc_label_gather(seqs, e_idx):
    B, L = seqs.shape
    N, K = e_idx.shape
    NW = 32
    rows_pw = N // NW
    nchunk = (rows_pw * K) // 16

    eidx_flat = e_idx.reshape(N * K)

    mesh = plsc.VectorSubcoreMesh(core_axis_name="c", subcore_axis_name="s")

    @functools.partial(
        pl.kernel,
        mesh=mesh,
        compiler_params=pltpu.CompilerParams(needs_layout_passes=False),
        out_type=jax.ShapeDtypeStruct((N * _KO,), jnp.int32),
        scratch_types=[
            pltpu.VMEM((L,), jnp.int32),
            pltpu.VMEM((rows_pw * K,), jnp.int32),
            pltpu.VMEM((rows_pw * _KO,), jnp.int32),
        ],
    )
    def k(seqs_hbm, eidx_hbm, out_hbm, seqs_v, eidx_v, g_v):
        wid = lax.axis_index("s") * 2 + lax.axis_index("c")
        row0 = wid * rows_pw
        b = row0 // L
        pltpu.sync_copy(seqs_hbm.at[b], seqs_v)
        pltpu.sync_copy(eidx_hbm.at[pl.ds(row0 * K, rows_pw * K)], eidx_v)

        def body(i, carry):
            e = eidx_v[pl.ds(i * 16, 16)]
            vals = plsc.load_gather(seqs_v, [e])
            pos = i * 16 + lax.broadcasted_iota(jnp.int32, (16,), 0)
            dest = (pos // K) * _KO + (pos % K)
            plsc.store_scatter(g_v, [dest], vals)
            return carry

        lax.fori_loop(0, nchunk, body, 0)
        pltpu.sync_copy(g_v, out_hbm.at[pl.ds(row0 * _KO, rows_pw * _KO)])

    return k(seqs, eidx_flat).reshape(N, _KO)


def _tc_body(etab_ref, g_ref, self_ref, mask_ref, gam_ref, bet_ref, t_ref,
             s_ref, out_ref):
    R = self_ref.shape[1]
    K = etab_ref.shape[2]
    AA = etab_ref.shape[3]
    A = self_ref.shape[2]

    Gb = g_ref[0]
    c400 = lax.broadcasted_iota(jnp.int32, (1, AA), 1) % A
    del t_ref

    acc = jnp.zeros((R, AA), jnp.float32)
    for j in range(1, K):
        gj = Gb[:, j][:, None]
        acc = acc + jnp.where(gj == c400, etab_ref[0, :, j, :], 0.0)

    pair = jnp.dot(acc, s_ref[...], preferred_element_type=jnp.float32)

    aa = self_ref[0] + pair
    mu = jnp.mean(aa, axis=1, keepdims=True)
    var = jnp.mean((aa - mu) ** 2, axis=1, keepdims=True)
    aa = (aa - mu) * lax.rsqrt(var + 1e-5) * gam_ref[0][None, :] + bet_ref[0][None, :]

    neg = -aa
    mx = jnp.max(neg, axis=1, keepdims=True)
    lse = jnp.log(jnp.sum(jnp.exp(neg - mx), axis=1, keepdims=True))
    s_oh = Gb[:, 0][:, None] == lax.broadcasted_iota(jnp.int32, (1, A), 1)
    negs = jnp.sum(jnp.where(s_oh, neg, 0.0), axis=1, keepdims=True)
    logp = negs - mx - lse
    out_ref[0, 0, :] = logp[:, 0] * mask_ref[0, 0, :]


def kernel(self_etab, etab, E_idx, seqs, x_mask, ln_gamma, ln_beta):
    B, L, K, AA = etab.shape
    A = self_etab.shape[-1]
    R = 128
    NB = L // R

    idx0 = lax.broadcasted_iota(jnp.int32, (B, L, 1), 1)
    e2 = jnp.concatenate([idx0, E_idx[:, :, 1:]], axis=2).reshape(B * L, K)

    G = _sc_label_gather(seqs.astype(jnp.int32), e2).reshape(B, L, _KO)

    x_mask_r = x_mask.reshape(B * NB, 1, R)
    gam = ln_gamma.reshape(1, A).astype(jnp.float32)
    bet = ln_beta.reshape(1, A).astype(jnp.float32)

    m_i = lax.broadcasted_iota(jnp.int32, (A, AA), 0)
    l_i = lax.broadcasted_iota(jnp.int32, (A, AA), 1)
    t_mat = (l_i % A == m_i).astype(jnp.float32)
    g_i = lax.broadcasted_iota(jnp.int32, (AA, A), 0)
    a_i = lax.broadcasted_iota(jnp.int32, (AA, A), 1)
    s_mat = (g_i // A == a_i).astype(jnp.float32)

    contrib = pl.pallas_call(
        _tc_body,
        grid=(B, NB),
        in_specs=[
            pl.BlockSpec((1, R, K, AA), lambda b, l: (b, l, 0, 0)),
            pl.BlockSpec((1, R, _KO), lambda b, l: (b, l, 0)),
            pl.BlockSpec((1, R, A), lambda b, l: (b, l, 0)),
            pl.BlockSpec((1, 1, R), lambda b, l: (b * NB + l, 0, 0)),
            pl.BlockSpec((1, A), lambda b, l: (0, 0)),
            pl.BlockSpec((1, A), lambda b, l: (0, 0)),
            pl.BlockSpec((A, AA), lambda b, l: (0, 0)),
            pl.BlockSpec((AA, A), lambda b, l: (0, 0)),
        ],
        out_specs=pl.BlockSpec((1, 1, R), lambda b, l: (b * NB + l, 0, 0)),
        out_shape=jax.ShapeDtypeStruct((B * NB, 1, R), jnp.float32),
    )(etab, G, self_etab, x_mask_r, gam, bet, t_mat, s_mat)

    contrib = contrib.reshape(B, L)
    n_res = jnp.sum(x_mask, axis=-1)
    nlpl = jnp.sum(contrib, axis=-1) / n_res
    return -jnp.mean(nlpl)


# speedup vs baseline: 8.2654x; 4.4384x over previous
import functools

import jax
import jax.numpy as jnp
from jax import lax
from jax.experimental import pallas as pl
from jax.experimental.pallas import tpu as pltpu
from jax.experimental.pallas import tpu_sc as plsc

_KO = 32


def _sc_label_gather(seqs, e_idx):
    B, L = seqs.shape
    K = e_idx.shape[2]
    N = B * L
    NW = 32
    rows_pw = N // NW
    nchunk = (rows_pw * K) // 16

    eidx_flat = e_idx.reshape(N * K)

    mesh = plsc.VectorSubcoreMesh(core_axis_name="c", subcore_axis_name="s")

    @functools.partial(
        pl.kernel,
        mesh=mesh,
        compiler_params=pltpu.CompilerParams(needs_layout_passes=False),
        out_type=jax.ShapeDtypeStruct((B, _KO, L), jnp.int32),
        scratch_types=[
            pltpu.VMEM((L,), jnp.int32),
            pltpu.VMEM((rows_pw * K,), jnp.int32),
            pltpu.VMEM((_KO, rows_pw), jnp.int32),
        ],
    )
    def k(seqs_hbm, eidx_hbm, out_hbm, seqs_v, eidx_v, g_v):
        wid = lax.axis_index("s") * 2 + lax.axis_index("c")
        row0 = wid * rows_pw
        b = row0 // L
        i0 = row0 % L
        pltpu.sync_copy(seqs_hbm.at[b], seqs_v)
        pltpu.sync_copy(eidx_hbm.at[pl.ds(row0 * K, rows_pw * K)], eidx_v)

        def body(i, carry):
            e = eidx_v[pl.ds(i * 16, 16)]
            vals = plsc.load_gather(seqs_v, [e])
            pos = i * 16 + lax.broadcasted_iota(jnp.int32, (16,), 0)
            plsc.store_scatter(g_v, [pos % K, pos // K], vals)
            return carry

        lax.fori_loop(0, nchunk, body, 0)
        pltpu.sync_copy(g_v, out_hbm.at[b, :, pl.ds(i0, rows_pw)])

    return k(seqs, eidx_flat)


def _tc_body(etab_ref, g_ref, self_ref, mask_ref, gam_ref, bet_ref, out_ref):
    K = etab_ref.shape[1]
    AA = etab_ref.shape[2]
    A = self_ref.shape[1]
    R = self_ref.shape[2]

    Gtb = g_ref[0]
    c_sub = lax.broadcasted_iota(jnp.int32, (AA, R), 0) % A

    acc = jnp.zeros((AA, R), jnp.float32)
    for j in range(1, K):
        gj = Gtb[j : j + 1, :]
        acc = acc + jnp.where(gj == c_sub, etab_ref[0, j], 0.0)

    pair = jnp.sum(acc.reshape(A, A, R), axis=1)

    aa = self_ref[0] + pair
    mu = jnp.mean(aa, axis=0, keepdims=True)
    var = jnp.mean((aa - mu) ** 2, axis=0, keepdims=True)
    aa = (aa - mu) * lax.rsqrt(var + 1e-5) * gam_ref[...] + bet_ref[...]

    neg = -aa
    mx = jnp.max(neg, axis=0, keepdims=True)
    lse = jnp.log(jnp.sum(jnp.exp(neg - mx), axis=0, keepdims=True))
    s_oh = Gtb[0:1, :] == lax.broadcasted_iota(jnp.int32, (A, R), 0)
    negs = jnp.sum(jnp.where(s_oh, neg, 0.0), axis=0, keepdims=True)
    logp = negs - mx - lse
    out_ref[0] = logp * mask_ref[0]


def kernel(self_etab, etab, E_idx, seqs, x_mask, ln_gamma, ln_beta):
    B, L, K, AA = etab.shape
    A = self_etab.shape[-1]
    R = 128
    NB = L // R

    idx0 = lax.broadcasted_iota(jnp.int32, (B, L, 1), 1)
    e2 = jnp.concatenate([idx0, E_idx[:, :, 1:]], axis=2)

    Gt = _sc_label_gather(seqs.astype(jnp.int32), e2)

    etab_t = jnp.transpose(etab, (0, 2, 3, 1))
    self_t = jnp.transpose(self_etab, (0, 2, 1))

    x_mask_r = x_mask.reshape(B * NB, 1, R)
    gam = jnp.broadcast_to(ln_gamma.astype(jnp.float32)[:, None], (A, R))
    bet = jnp.broadcast_to(ln_beta.astype(jnp.float32)[:, None], (A, R))

    contrib = pl.pallas_call(
        _tc_body,
        grid=(B, NB),
        in_specs=[
            pl.BlockSpec((1, K, AA, R), lambda b, l: (b, 0, 0, l)),
            pl.BlockSpec((1, _KO, R), lambda b, l: (b, 0, l)),
            pl.BlockSpec((1, A, R), lambda b, l: (b, 0, l)),
            pl.BlockSpec((1, 1, R), lambda b, l: (b * NB + l, 0, 0)),
            pl.BlockSpec((A, R), lambda b, l: (0, 0)),
            pl.BlockSpec((A, R), lambda b, l: (0, 0)),
        ],
        out_specs=pl.BlockSpec((1, 1, R), lambda b, l: (b * NB + l, 0, 0)),
        out_shape=jax.ShapeDtypeStruct((B * NB, 1, R), jnp.float32),
    )(etab_t, Gt, self_t, x_mask_r, gam, bet)

    contrib = contrib.reshape(B, L)
    n_res = jnp.sum(x_mask, axis=-1)
    nlpl = jnp.sum(contrib, axis=-1) / n_res
    return -jnp.mean(nlpl)


# speedup vs baseline: 8.6739x; 1.0494x over previous
import functools

import jax
import jax.numpy as jnp
from jax import lax
from jax.experimental import pallas as pl
from jax.experimental.pallas import tpu as pltpu
from jax.experimental.pallas import tpu_sc as plsc

_KO = 32


def _sc_label_gather(seqs, e_idx):
    B, L = seqs.shape
    K = e_idx.shape[2]
    N = B * L
    NW = 32
    rows_pw = N // NW
    nchunk = (rows_pw * K) // 16

    eidx_flat = e_idx.reshape(N * K)

    mesh = plsc.VectorSubcoreMesh(core_axis_name="c", subcore_axis_name="s")

    @functools.partial(
        pl.kernel,
        mesh=mesh,
        compiler_params=pltpu.CompilerParams(needs_layout_passes=False),
        out_type=jax.ShapeDtypeStruct((B, _KO, L), jnp.int32),
        scratch_types=[
            pltpu.VMEM((L,), jnp.int32),
            pltpu.VMEM((rows_pw * K,), jnp.int32),
            pltpu.VMEM((_KO, rows_pw), jnp.int32),
        ],
    )
    def k(seqs_hbm, eidx_hbm, out_hbm, seqs_v, eidx_v, g_v):
        wid = lax.axis_index("s") * 2 + lax.axis_index("c")
        row0 = wid * rows_pw
        b = row0 // L
        i0 = row0 % L
        pltpu.sync_copy(seqs_hbm.at[b], seqs_v)
        pltpu.sync_copy(eidx_hbm.at[pl.ds(row0 * K, rows_pw * K)], eidx_v)

        def body(i, carry):
            e = eidx_v[pl.ds(i * 16, 16)]
            pos = i * 16 + lax.broadcasted_iota(jnp.int32, (16,), 0)
            j = pos % K
            il = pos // K
            e = jnp.where(j == 0, i0 + il, e)
            vals = plsc.load_gather(seqs_v, [e])
            plsc.store_scatter(g_v, [j, il], vals)
            return carry

        lax.fori_loop(0, nchunk, body, 0)
        pltpu.sync_copy(g_v, out_hbm.at[b, :, pl.ds(i0, rows_pw)])

    return k(seqs, eidx_flat)


def _tc_body(etab_ref, g_ref, self_ref, mask_ref, gam_ref, bet_ref, out_ref):
    K = etab_ref.shape[1]
    AA = etab_ref.shape[2]
    A = self_ref.shape[1]
    R = self_ref.shape[2]

    Gtb = g_ref[0]
    c_sub = lax.broadcasted_iota(jnp.int32, (AA, R), 0) % A

    acc = jnp.zeros((AA, R), jnp.float32)
    for j in range(1, K):
        gj = Gtb[j : j + 1, :]
        acc = acc + jnp.where(gj == c_sub, etab_ref[0, j], 0.0)

    pair = jnp.sum(acc.reshape(A, A, R), axis=1)

    aa = self_ref[0] + pair
    mu = jnp.mean(aa, axis=0, keepdims=True)
    var = jnp.mean((aa - mu) ** 2, axis=0, keepdims=True)
    aa = (aa - mu) * lax.rsqrt(var + 1e-5) * gam_ref[...] + bet_ref[...]

    neg = -aa
    mx = jnp.max(neg, axis=0, keepdims=True)
    lse = jnp.log(jnp.sum(jnp.exp(neg - mx), axis=0, keepdims=True))
    s_oh = Gtb[0:1, :] == lax.broadcasted_iota(jnp.int32, (A, R), 0)
    negs = jnp.sum(jnp.where(s_oh, neg, 0.0), axis=0, keepdims=True)
    logp = negs - mx - lse
    out_ref[0] = logp * mask_ref[0]


def kernel(self_etab, etab, E_idx, seqs, x_mask, ln_gamma, ln_beta):
    B, L, K, AA = etab.shape
    A = self_etab.shape[-1]
    R = 512
    NB = L // R

    Gt = _sc_label_gather(seqs.astype(jnp.int32), E_idx)

    etab_t = jnp.transpose(etab, (0, 2, 3, 1))
    self_t = jnp.transpose(self_etab, (0, 2, 1))

    x_mask_r = x_mask.reshape(B * NB, 1, R)
    gam = jnp.broadcast_to(ln_gamma.astype(jnp.float32)[:, None], (A, R))
    bet = jnp.broadcast_to(ln_beta.astype(jnp.float32)[:, None], (A, R))

    contrib = pl.pallas_call(
        _tc_body,
        grid=(B, NB),
        in_specs=[
            pl.BlockSpec((1, K, AA, R), lambda b, l: (b, 0, 0, l)),
            pl.BlockSpec((1, _KO, R), lambda b, l: (b, 0, l)),
            pl.BlockSpec((1, A, R), lambda b, l: (b, 0, l)),
            pl.BlockSpec((1, 1, R), lambda b, l: (b * NB + l, 0, 0)),
            pl.BlockSpec((A, R), lambda b, l: (0, 0)),
            pl.BlockSpec((A, R), lambda b, l: (0, 0)),
        ],
        out_specs=pl.BlockSpec((1, 1, R), lambda b, l: (b * NB + l, 0, 0)),
        out_shape=jax.ShapeDtypeStruct((B * NB, 1, R), jnp.float32),
    )(etab_t, Gt, self_t, x_mask_r, gam, bet)

    contrib = contrib.reshape(B, L)
    n_res = jnp.sum(x_mask, axis=-1)
    nlpl = jnp.sum(contrib, axis=-1) / n_res
    return -jnp.mean(nlpl)
